# baseline (speedup 1.0000x reference)
import functools

import jax
import jax.numpy as jnp
from jax import lax
from jax.experimental import pallas as pl
from jax.experimental.pallas import tpu as pltpu
from jax.experimental.pallas import tpu_sc as plsc

NUM_WORKERS = 32
NUM_CORES = 2


@jax.jit
def _embed(table, ids):
    bsz, hist = ids.shape
    d = table.shape[1]
    rows_per_w = bsz // NUM_WORKERS
    n2 = rows_per_w // 2

    mesh = plsc.VectorSubcoreMesh(core_axis_name="c", subcore_axis_name="s")

    @functools.partial(
        pl.kernel,
        out_type=jax.ShapeDtypeStruct((bsz, hist, d), jnp.float32),
        mesh=mesh,
        scratch_types=[
            pltpu.VMEM((rows_per_w, hist), jnp.int32),
            pltpu.VMEM((hist, d), jnp.float32),
            pltpu.VMEM((hist, d), jnp.float32),
            pltpu.SemaphoreType.DMA,
            pltpu.SemaphoreType.DMA,
            pltpu.SemaphoreType.DMA,
            pltpu.SemaphoreType.DMA,
        ],
        compiler_params=pltpu.CompilerParams(use_tc_tiling_on_sc=False),
    )
    def k(table_hbm, ids_hbm, out_hbm, idx_v, r0, r1, sg0, sg1, ss0, ss1):
        wid = lax.axis_index("s") * NUM_CORES + lax.axis_index("c")
        base = wid * rows_per_w

        def fire_gather(b, rows, sem):
            pltpu.async_copy(table_hbm.at[idx_v.at[b]], rows, sem)

        def wait_gather(rows, sem):
            pltpu.make_async_copy(table_hbm.at[idx_v.at[0]], rows, sem).wait()

        def fire_store(rows, b, sem):
            pltpu.async_copy(rows, out_hbm.at[base + b], sem)

        def wait_store(rows, b, sem):
            pltpu.make_async_copy(rows, out_hbm.at[base + b], sem).wait()

        pltpu.sync_copy(ids_hbm.at[pl.ds(base, rows_per_w)], idx_v)

        fire_gather(0, r0, sg0)

        def body(g2, carry):
            ba = 2 * g2
            bb = ba + 1

            @pl.when(g2 > 0)
            def _():
                wait_store(r1, ba - 1, ss1)

            fire_gather(bb, r1, sg1)
            wait_gather(r0, sg0)
            fire_store(r0, ba, ss0)

            wait_store(r0, ba, ss0)

            @pl.when(bb + 1 < rows_per_w)
            def _():
                fire_gather(bb + 1, r0, sg0)

            wait_gather(r1, sg1)
            fire_store(r1, bb, ss1)
            return carry

        lax.fori_loop(0, n2, body, 0)
        wait_store(r1, rows_per_w - 1, ss1)

    return k(table, ids)


def kernel(input_ids, embedding_table):
    return _embed(embedding_table, input_ids.astype(jnp.int32))
